# baseline (speedup 1.0000x reference)
import functools

import jax
import jax.numpy as jnp
from jax import lax
from jax.experimental import pallas as pl
from jax.experimental.pallas import tpu as pltpu
from jax.experimental.pallas import tpu_sc as plsc

B = 4096
L = 200
DIM = 64
ACTION_SPACE = 20
VOCAB = 1000
VP = 1024
NG = VP // 128

NC = 2
NS = 16
NW = NC * NS
RCHUNK = 32
LPAD = 211

NSPLIT = 1
BH = B // NSPLIT
CPW = BH // RCHUNK // NW


def _hist_body(text_hbm, counts_hbm, tok_v, cnt_v, sem_in, sem_out):
    wid = lax.axis_index("s") * NC + lax.axis_index("c")
    lane = lax.iota(jnp.int32, 16)
    ones = jnp.full((16,), 1.0, dtype=jnp.float32)
    zeros = jnp.zeros((16,), dtype=jnp.float32)
    row_ids = [lane + 16 * h for h in range(RCHUNK // 16)]

    def start_in(c):
        row0 = (wid * CPW + c) * RCHUNK
        return pltpu.async_copy(
            text_hbm.at[pl.ds(row0, RCHUNK)],
            tok_v.at[c % 2, :, pl.ds(0, L)], sem_in.at[c % 2])

    def start_out(c):
        row0 = (wid * CPW + c) * RCHUNK
        return [pltpu.async_copy(
                    cnt_v.at[c % 2, :, pl.ds(128 * g, 128)],
                    counts_hbm.at[g, pl.ds(row0, RCHUNK)], sem_out.at[c % 2])
                for g in range(NG)]

    din = {0: start_in(0)}
    dout = {}
    for c in range(CPW):
        if c + 1 < CPW:
            din[c + 1] = start_in(c + 1)
        din.pop(c).wait()
        if c >= 2:
            for d in dout.pop(c - 2):
                d.wait()

        def _zero(r, carry):
            for k in range(VP // 16):
                cnt_v[c % 2, r, pl.ds(16 * k, 16)] = zeros
            return carry
        lax.fori_loop(0, RCHUNK, _zero, 0)

        def _scat(i, lcol):
            for _ in range(4):
                for h in range(RCHUNK // 16):
                    tok = plsc.load_gather(tok_v.at[c % 2],
                                           [row_ids[h], lcol])
                    plsc.addupdate_scatter(cnt_v.at[c % 2],
                                           [row_ids[h], tok], ones)
                lcol = lcol + 1
            return lcol
        lax.fori_loop(0, L // 4, _scat, jnp.zeros((16,), jnp.int32))

        dout[c] = start_out(c)
    for c in sorted(dout):
        for d in dout.pop(c):
            d.wait()


@functools.partial(
    pl.kernel,
    out_type=jax.ShapeDtypeStruct((NG, BH, 128), jnp.float32),
    mesh=plsc.VectorSubcoreMesh(
        core_axis_name="c", subcore_axis_name="s", num_cores=NC, num_subcores=NS
    ),
    scratch_types=[
        pltpu.VMEM((2, RCHUNK, LPAD), jnp.int32),
        pltpu.VMEM((2, RCHUNK, VP), jnp.float32),
        pltpu.SemaphoreType.DMA((2,)),
        pltpu.SemaphoreType.DMA((2,)),
    ],
    compiler_params=pltpu.CompilerParams(
        use_tc_tiling_on_sc=False, needs_layout_passes=False
    ),
)
def _histogram(text_hbm, counts_hbm, tok_v, cnt_v, sem_in, sem_out):
    _hist_body(text_hbm, counts_hbm, tok_v, cnt_v, sem_in, sem_out)


BB = 256


def _dense_body(cnt_ref, emb_ref, pw_ref, pb_ref, lw_ref, lb_ref, vw_ref,
                vb_ref, out_ref):
    hi = jax.lax.Precision.DEFAULT
    pooled = jnp.dot(cnt_ref[0], emb_ref[0], precision=hi,
                     preferred_element_type=jnp.float32)
    for g in range(1, NG):
        pooled += jnp.dot(cnt_ref[g], emb_ref[g], precision=hi,
                          preferred_element_type=jnp.float32)
    pooled *= 1.0 / L
    h = jnp.dot(pooled, pw_ref[...], precision=hi,
                preferred_element_type=jnp.float32) + pb_ref[...]
    lg = jnp.dot(h, lw_ref[...], precision=hi,
                 preferred_element_type=jnp.float32) + lb_ref[...]
    val = jnp.dot(h, vw_ref[...], precision=hi,
                  preferred_element_type=jnp.float32) + vb_ref[...]
    pad = jnp.zeros((BB, 128 - DIM - ACTION_SPACE - 1), jnp.float32)
    out_ref[...] = jnp.concatenate([h, lg, val, pad], axis=1)


def _dense(counts, embg, proj_w, proj_b, logit_w, logit_b, value_w, value_b):
    grid = (BH // BB,)
    return pl.pallas_call(
        _dense_body,
        grid=grid,
        in_specs=[
            pl.BlockSpec((NG, BB, 128), lambda i: (0, i, 0)),
            pl.BlockSpec((NG, 128, DIM), lambda i: (0, 0, 0)),
            pl.BlockSpec((DIM, DIM), lambda i: (0, 0)),
            pl.BlockSpec((1, DIM), lambda i: (0, 0)),
            pl.BlockSpec((DIM, ACTION_SPACE), lambda i: (0, 0)),
            pl.BlockSpec((1, ACTION_SPACE), lambda i: (0, 0)),
            pl.BlockSpec((DIM, 1), lambda i: (0, 0)),
            pl.BlockSpec((1, 1), lambda i: (0, 0)),
        ],
        out_specs=pl.BlockSpec((BB, 128), lambda i: (i, 0)),
        out_shape=jax.ShapeDtypeStruct((BH, 128), jnp.float32),
    )(counts, embg, proj_w, proj_b, logit_w, logit_b, value_w, value_b)


def kernel(text, emb, proj_w, proj_b, logit_w, logit_b, value_w, value_b):
    ti = text.astype(jnp.int32)
    embp = jnp.pad(emb, ((0, VP - VOCAB), (0, 0)))
    embg = embp.reshape(NG, 128, DIM)
    pb = proj_b.reshape(1, DIM)
    lb = logit_b.reshape(1, ACTION_SPACE)
    vb = value_b.reshape(1, 1)

    counts = _histogram(ti)
    packed = _dense(counts, embg, proj_w, pb, logit_w, lb, value_w, vb)
    h = packed[:, :DIM]
    logits = packed[:, DIM:DIM + ACTION_SPACE]
    value = packed[:, DIM + ACTION_SPACE:DIM + ACTION_SPACE + 1]

    hidden = jnp.broadcast_to(h[:, None, :], (B, L, DIM))
    custom_key = jnp.zeros((2, 3), dtype=jnp.float32)
    return (hidden, logits, value, custom_key)


# speedup vs baseline: 1.0147x; 1.0147x over previous
import functools

import jax
import jax.numpy as jnp
from jax import lax
from jax.experimental import pallas as pl
from jax.experimental.pallas import tpu as pltpu
from jax.experimental.pallas import tpu_sc as plsc

B = 4096
L = 200
DIM = 64
ACTION_SPACE = 20
VOCAB = 1000
VP = 1024
NG = VP // 128

NC = 2
NS = 16
NW = NC * NS
RCHUNK = 32
LPAD = 211

NSPLIT = 1
BH = B // NSPLIT
CPW = BH // RCHUNK // NW


def _hist_body(text_hbm, counts_hbm, tok_v, cnt_v, sem_in, sem_out):
    wid = lax.axis_index("s") * NC + lax.axis_index("c")
    lane = lax.iota(jnp.int32, 16)
    ones = jnp.full((16,), 1.0, dtype=jnp.float32)
    zeros = jnp.zeros((16,), dtype=jnp.float32)
    row_ids = [lane + 16 * h for h in range(RCHUNK // 16)]

    def start_in(c):
        row0 = (wid * CPW + c) * RCHUNK
        return pltpu.async_copy(
            text_hbm.at[pl.ds(row0, RCHUNK)],
            tok_v.at[c % 2, :, pl.ds(0, L)], sem_in.at[c % 2])

    def start_out(c):
        row0 = (wid * CPW + c) * RCHUNK
        return [pltpu.async_copy(
                    cnt_v.at[c % 2, :, pl.ds(128 * g, 128)],
                    counts_hbm.at[g, pl.ds(row0, RCHUNK)], sem_out.at[c % 2])
                for g in range(NG)]

    din = {0: start_in(0)}
    dout = {}
    for c in range(CPW):
        if c + 1 < CPW:
            din[c + 1] = start_in(c + 1)
        din.pop(c).wait()
        if c >= 2:
            for d in dout.pop(c - 2):
                d.wait()

        def _zero(r, carry):
            for k in range(VP // 16):
                cnt_v[c % 2, r, pl.ds(16 * k, 16)] = zeros
            return carry
        lax.fori_loop(0, RCHUNK, _zero, 0)

        def _scat(i, lcol):
            for _ in range(4):
                for h in range(RCHUNK // 16):
                    tok = plsc.load_gather(tok_v.at[c % 2],
                                           [row_ids[h], lcol])
                    plsc.addupdate_scatter(cnt_v.at[c % 2],
                                           [row_ids[h], tok], ones)
                lcol = lcol + 1
            return lcol
        lax.fori_loop(0, L // 4, _scat, jnp.zeros((16,), jnp.int32))

        dout[c] = start_out(c)
    for c in sorted(dout):
        for d in dout.pop(c):
            d.wait()


@functools.partial(
    pl.kernel,
    out_type=jax.ShapeDtypeStruct((NG, BH, 128), jnp.float32),
    mesh=plsc.VectorSubcoreMesh(
        core_axis_name="c", subcore_axis_name="s", num_cores=NC, num_subcores=NS
    ),
    scratch_types=[
        pltpu.VMEM((2, RCHUNK, LPAD), jnp.int32),
        pltpu.VMEM((2, RCHUNK, VP), jnp.float32),
        pltpu.SemaphoreType.DMA((2,)),
        pltpu.SemaphoreType.DMA((2,)),
    ],
    compiler_params=pltpu.CompilerParams(
        use_tc_tiling_on_sc=False, needs_layout_passes=False
    ),
)
def _histogram(text_hbm, counts_hbm, tok_v, cnt_v, sem_in, sem_out):
    _hist_body(text_hbm, counts_hbm, tok_v, cnt_v, sem_in, sem_out)


BB = 256


def _dense_body(cnt_ref, emb_ref, pw_ref, pb_ref, lw_ref, lb_ref, vw_ref,
                vb_ref, h_ref, lg_ref, val_ref):
    hi = jax.lax.Precision.DEFAULT
    pooled = jnp.dot(cnt_ref[0], emb_ref[0], precision=hi,
                     preferred_element_type=jnp.float32)
    for g in range(1, NG):
        pooled += jnp.dot(cnt_ref[g], emb_ref[g], precision=hi,
                          preferred_element_type=jnp.float32)
    pooled *= 1.0 / L
    h = jnp.dot(pooled, pw_ref[...], precision=hi,
                preferred_element_type=jnp.float32) + pb_ref[...]
    h_ref[...] = h
    lg_ref[...] = jnp.dot(h, lw_ref[...], precision=hi,
                          preferred_element_type=jnp.float32) + lb_ref[...]
    val_ref[...] = jnp.dot(h, vw_ref[...], precision=hi,
                           preferred_element_type=jnp.float32) + vb_ref[...]


def _dense(counts, embg, proj_w, proj_b, logit_w, logit_b, value_w, value_b):
    grid = (BH // BB,)
    return pl.pallas_call(
        _dense_body,
        grid=grid,
        in_specs=[
            pl.BlockSpec((NG, BB, 128), lambda i: (0, i, 0)),
            pl.BlockSpec((NG, 128, DIM), lambda i: (0, 0, 0)),
            pl.BlockSpec((DIM, DIM), lambda i: (0, 0)),
            pl.BlockSpec((1, DIM), lambda i: (0, 0)),
            pl.BlockSpec((DIM, ACTION_SPACE), lambda i: (0, 0)),
            pl.BlockSpec((1, ACTION_SPACE), lambda i: (0, 0)),
            pl.BlockSpec((DIM, 1), lambda i: (0, 0)),
            pl.BlockSpec((1, 1), lambda i: (0, 0)),
        ],
        out_specs=[
            pl.BlockSpec((BB, DIM), lambda i: (i, 0)),
            pl.BlockSpec((BB, ACTION_SPACE), lambda i: (i, 0)),
            pl.BlockSpec((BB, 1), lambda i: (i, 0)),
        ],
        out_shape=[
            jax.ShapeDtypeStruct((BH, DIM), jnp.float32),
            jax.ShapeDtypeStruct((BH, ACTION_SPACE), jnp.float32),
            jax.ShapeDtypeStruct((BH, 1), jnp.float32),
        ],
    )(counts, embg, proj_w, proj_b, logit_w, logit_b, value_w, value_b)


def kernel(text, emb, proj_w, proj_b, logit_w, logit_b, value_w, value_b):
    ti = text.astype(jnp.int32)
    embp = jnp.pad(emb, ((0, VP - VOCAB), (0, 0)))
    embg = embp.reshape(NG, 128, DIM)
    pb = proj_b.reshape(1, DIM)
    lb = logit_b.reshape(1, ACTION_SPACE)
    vb = value_b.reshape(1, 1)

    counts = _histogram(ti)
    h, logits, value = _dense(counts, embg, proj_w, pb, logit_w, lb,
                              value_w, vb)

    hidden = jnp.broadcast_to(h[:, None, :], (B, L, DIM))
    custom_key = jnp.zeros((2, 3), dtype=jnp.float32)
    return (hidden, logits, value, custom_key)


# speedup vs baseline: 1.0534x; 1.0381x over previous
import functools

import jax
import jax.numpy as jnp
from jax import lax
from jax.experimental import pallas as pl
from jax.experimental.pallas import tpu as pltpu
from jax.experimental.pallas import tpu_sc as plsc

B = 4096
L = 200
DIM = 64
ACTION_SPACE = 20
VOCAB = 1000
VP = 1024
NG = VP // 128

NC = 2
NS = 16
NW = NC * NS
RCHUNK = 32
LPAD = 211

NSPLIT = 1
BH = B // NSPLIT
CPW = BH // RCHUNK // NW


def _hist_body(text_hbm, counts_hbm, tok_v, cnt_v, sem_in, sem_out):
    wid = lax.axis_index("s") * NC + lax.axis_index("c")
    lane = lax.iota(jnp.int32, 16)
    ones = jnp.full((16,), 1.0, dtype=jnp.float32)
    zeros = jnp.zeros((16,), dtype=jnp.float32)
    row_ids = [lane + 16 * h for h in range(RCHUNK // 16)]

    def start_in(c):
        row0 = (wid * CPW + c) * RCHUNK
        return pltpu.async_copy(
            text_hbm.at[pl.ds(row0, RCHUNK)],
            tok_v.at[c % 2, :, pl.ds(0, L)], sem_in.at[c % 2])

    def start_out(c):
        row0 = (wid * CPW + c) * RCHUNK
        return [pltpu.async_copy(
                    cnt_v.at[c % 2, :, pl.ds(128 * g, 128)],
                    counts_hbm.at[g, pl.ds(row0, RCHUNK)], sem_out.at[c % 2])
                for g in range(NG)]

    din = {0: start_in(0)}
    dout = {}
    for c in range(CPW):
        if c + 1 < CPW:
            din[c + 1] = start_in(c + 1)
        din.pop(c).wait()
        if c >= 2:
            for d in dout.pop(c - 2):
                d.wait()

        def _zero(r, carry):
            for k in range(VP // 16):
                cnt_v[c % 2, r, pl.ds(16 * k, 16)] = zeros
            return carry
        lax.fori_loop(0, RCHUNK, _zero, 0)

        def _scat(i, lcol):
            for _ in range(4):
                for h in range(RCHUNK // 16):
                    tok = plsc.load_gather(tok_v.at[c % 2],
                                           [row_ids[h], lcol])
                    plsc.addupdate_scatter(cnt_v.at[c % 2],
                                           [row_ids[h], tok], ones)
                lcol = lcol + 1
            return lcol
        lax.fori_loop(0, L // 4, _scat, jnp.zeros((16,), jnp.int32))

        dout[c] = start_out(c)
    for c in sorted(dout):
        for d in dout.pop(c):
            d.wait()


@functools.partial(
    pl.kernel,
    out_type=jax.ShapeDtypeStruct((NG, BH, 128), jnp.float32),
    mesh=plsc.VectorSubcoreMesh(
        core_axis_name="c", subcore_axis_name="s", num_cores=NC, num_subcores=NS
    ),
    scratch_types=[
        pltpu.VMEM((2, RCHUNK, LPAD), jnp.int32),
        pltpu.VMEM((2, RCHUNK, VP), jnp.float32),
        pltpu.SemaphoreType.DMA((2,)),
        pltpu.SemaphoreType.DMA((2,)),
    ],
    compiler_params=pltpu.CompilerParams(
        use_tc_tiling_on_sc=False, needs_layout_passes=False
    ),
)
def _histogram(text_hbm, counts_hbm, tok_v, cnt_v, sem_in, sem_out):
    _hist_body(text_hbm, counts_hbm, tok_v, cnt_v, sem_in, sem_out)


BB = 512


def _dense_body(cnt_ref, emb_ref, pw_ref, pb_ref, lw_ref, lb_ref, vw_ref,
                vb_ref, h_ref, lg_ref, val_ref):
    hi = jax.lax.Precision.DEFAULT
    pooled = jnp.dot(cnt_ref[0], emb_ref[0], precision=hi,
                     preferred_element_type=jnp.float32)
    for g in range(1, NG):
        pooled += jnp.dot(cnt_ref[g], emb_ref[g], precision=hi,
                          preferred_element_type=jnp.float32)
    pooled *= 1.0 / L
    h = jnp.dot(pooled, pw_ref[...], precision=hi,
                preferred_element_type=jnp.float32) + pb_ref[...]
    h_ref[...] = h
    lg_ref[...] = jnp.dot(h, lw_ref[...], precision=hi,
                          preferred_element_type=jnp.float32) + lb_ref[...]
    val_ref[...] = jnp.dot(h, vw_ref[...], precision=hi,
                           preferred_element_type=jnp.float32) + vb_ref[...]


def _dense(counts, embg, proj_w, proj_b, logit_w, logit_b, value_w, value_b):
    grid = (BH // BB,)
    return pl.pallas_call(
        _dense_body,
        grid=grid,
        in_specs=[
            pl.BlockSpec((NG, BB, 128), lambda i: (0, i, 0)),
            pl.BlockSpec((NG, 128, DIM), lambda i: (0, 0, 0)),
            pl.BlockSpec((DIM, DIM), lambda i: (0, 0)),
            pl.BlockSpec((1, DIM), lambda i: (0, 0)),
            pl.BlockSpec((DIM, ACTION_SPACE), lambda i: (0, 0)),
            pl.BlockSpec((1, ACTION_SPACE), lambda i: (0, 0)),
            pl.BlockSpec((DIM, 1), lambda i: (0, 0)),
            pl.BlockSpec((1, 1), lambda i: (0, 0)),
        ],
        out_specs=[
            pl.BlockSpec((BB, DIM), lambda i: (i, 0)),
            pl.BlockSpec((BB, ACTION_SPACE), lambda i: (i, 0)),
            pl.BlockSpec((BB, 1), lambda i: (i, 0)),
        ],
        out_shape=[
            jax.ShapeDtypeStruct((BH, DIM), jnp.float32),
            jax.ShapeDtypeStruct((BH, ACTION_SPACE), jnp.float32),
            jax.ShapeDtypeStruct((BH, 1), jnp.float32),
        ],
    )(counts, embg, proj_w, proj_b, logit_w, logit_b, value_w, value_b)


def kernel(text, emb, proj_w, proj_b, logit_w, logit_b, value_w, value_b):
    ti = text.astype(jnp.int32)
    embp = jnp.pad(emb, ((0, VP - VOCAB), (0, 0)))
    embg = embp.reshape(NG, 128, DIM)
    pb = proj_b.reshape(1, DIM)
    lb = logit_b.reshape(1, ACTION_SPACE)
    vb = value_b.reshape(1, 1)

    counts = _histogram(ti)
    h, logits, value = _dense(counts, embg, proj_w, pb, logit_w, lb,
                              value_w, vb)

    hidden = jnp.broadcast_to(h[:, None, :], (B, L, DIM))
    custom_key = jnp.zeros((2, 3), dtype=jnp.float32)
    return (hidden, logits, value, custom_key)
